# Initial kernel scaffold; baseline (speedup 1.0000x reference)
import jax
import jax.numpy as jnp
from jax.experimental import pallas as pl


def kernel(node_feats, adj_dict, W0, W1, gamma0, beta0, gamma1, beta1):
    raise NotImplementedError("write your pallas kernel here")


# speedup vs baseline: 1.3229x; 1.3229x over previous
import functools

import jax
import jax.numpy as jnp
from jax.experimental import pallas as pl
from jax.experimental.pallas import tpu as pltpu

N = 4096
D = 128
BM = 512


def _gcn_kernel(x_full_ref, w_ref, adj_ref, x_blk_ref, gamma_ref, beta_ref,
                out_ref, hproj_ref):
    i = pl.program_id(1)

    @pl.when(i == 0)
    def _():
        hproj_ref[...] = jax.lax.dot_general(
            x_full_ref[0], w_ref[0],
            dimension_numbers=(((1,), (1,)), ((), ())),
            preferred_element_type=jnp.float32,
        )

    agg = jnp.dot(adj_ref[0], hproj_ref[...],
                  preferred_element_type=jnp.float32)
    h = agg + x_blk_ref[0]
    mu = jnp.mean(h, axis=-1, keepdims=True)
    c = h - mu
    var = jnp.mean(c * c, axis=-1, keepdims=True)
    out_ref[0] = c * jax.lax.rsqrt(var + 1e-5) * gamma_ref[0] + beta_ref[0]


@jax.jit
def _gcn(node_feats, adj_dict, Ws, gammas, betas):
    grid = (2, N // BM)
    out = pl.pallas_call(
        _gcn_kernel,
        grid=grid,
        in_specs=[
            pl.BlockSpec((1, N, D), lambda t, i: (t, 0, 0)),
            pl.BlockSpec((1, D, D), lambda t, i: (t, 0, 0)),
            pl.BlockSpec((1, BM, N), lambda t, i: (t, i, 0)),
            pl.BlockSpec((1, BM, D), lambda t, i: (t, i, 0)),
            pl.BlockSpec((1, 1, D), lambda t, i: (t, 0, 0)),
            pl.BlockSpec((1, 1, D), lambda t, i: (t, 0, 0)),
        ],
        out_specs=pl.BlockSpec((1, BM, D), lambda t, i: (t, i, 0)),
        out_shape=jax.ShapeDtypeStruct((2, N, D), jnp.float32),
        scratch_shapes=[pltpu.VMEM((N, D), jnp.float32)],
        compiler_params=pltpu.CompilerParams(
            dimension_semantics=("parallel", "arbitrary"),
        ),
    )(node_feats, Ws, adj_dict, node_feats, gammas, betas)
    return out.reshape(2 * N, D)


def kernel(node_feats, adj_dict, W0, W1, gamma0, beta0, gamma1, beta1):
    Ws = jnp.stack((W0, W1))
    gammas = jnp.stack((gamma0, gamma1)).reshape(2, 1, D)
    betas = jnp.stack((beta0, beta1)).reshape(2, 1, D)
    return _gcn(node_feats, adj_dict, Ws, gammas, betas)
